# baseline (speedup 1.0000x reference)
import functools

import jax
import jax.numpy as jnp
import numpy as np
from jax import lax
from jax.experimental import pallas as pl
from jax.experimental.pallas import tpu as pltpu
from jax.experimental.pallas import tpu_sc as plsc

_V = 1000000
_D = 128
_B = 4096
_L = 50

_NC = 2
_NS = 16
_NW = _NC * _NS
_BPW = _B // _NW
_IPC = _L
_CHUNKS = _BPW


def _pool_body(idx_hbm, rid_hbm, zero_hbm, table_hbm, pooled_hbm,
               idx_v, rid_v, buf0, buf1, pooled_v, shared_acc,
               sem_g0, sem_g1):
    cid = lax.axis_index("c")
    sid = lax.axis_index("s")
    wid = sid * _NC + cid
    pltpu.sync_copy(idx_hbm.at[pl.ds(wid * _BPW, _BPW)], idx_v)
    pltpu.sync_copy(rid_hbm.at[sid], rid_v)
    pltpu.sync_copy(zero_hbm, pooled_v)
    pltpu.sync_copy(pooled_v, shared_acc.at[pl.ds(sid * _BPW, _BPW)])

    bufs = (buf0, buf1)
    gsems = (sem_g0, sem_g1)

    def _gather(c, p):
        return pltpu.async_copy(table_hbm.at[idx_v.at[c]], bufs[p], gsems[p])

    _gather(0, 0)
    _gather(1, 1)

    def pair_body(c2, carry):
        for p in range(2):
            c = c2 * 2 + p
            pltpu.make_async_copy(
                table_hbm.at[idx_v.at[c]], bufs[p], gsems[p]
            ).wait()
            pltpu.sync_copy(bufs[p], shared_acc.at[rid_v.at[c]], add=True)

            @pl.when(c + 2 < _CHUNKS)
            def _():
                _gather(c + 2, p)

        return carry

    lax.fori_loop(0, _CHUNKS // 2, pair_body, 0)
    pltpu.sync_copy(shared_acc.at[pl.ds(sid * _BPW, _BPW)], pooled_v)
    pltpu.sync_copy(pooled_v, pooled_hbm.at[pl.ds(wid * _BPW, _BPW)])


_pool = functools.partial(
    pl.kernel,
    mesh=plsc.VectorSubcoreMesh(core_axis_name="c", subcore_axis_name="s"),
    out_type=jax.ShapeDtypeStruct((_B, _D), jnp.float32),
    scratch_types=[
        pltpu.VMEM((_CHUNKS, _IPC), jnp.int32),
        pltpu.VMEM((_CHUNKS, _IPC), jnp.int32),
        pltpu.VMEM((_IPC, _D), jnp.float32),
        pltpu.VMEM((_IPC, _D), jnp.float32),
        pltpu.VMEM((_BPW, _D), jnp.float32),
        pltpu.VMEM_SHARED((_NS * _BPW, _D), jnp.float32),
        pltpu.SemaphoreType.DMA,
        pltpu.SemaphoreType.DMA,
    ],
)(_pool_body)

_RID = (
    np.broadcast_to(np.arange(_CHUNKS, dtype=np.int32)[:, None], (_CHUNKS, _IPC))
    [None, :, :]
    + (np.arange(_NS, dtype=np.int32) * _BPW)[:, None, None]
).copy()


def _linear_body(pooled_ref, wt_ref, bias_ref, out_ref):
    out_ref[...] = (
        jnp.dot(pooled_ref[...], wt_ref[...], preferred_element_type=jnp.float32)
        + bias_ref[...]
    )


def kernel(s, table, W, b):
    flat_idx = s.astype(jnp.int32)
    rid = jnp.asarray(_RID)
    zeros = jnp.zeros((_BPW, _D), jnp.float32)
    pooled = _pool(flat_idx, rid, zeros, table)
    out = pl.pallas_call(
        _linear_body,
        out_shape=jax.ShapeDtypeStruct((_B, 2), jnp.float32),
    )(pooled, W.T.astype(jnp.float32), (_L * b).reshape(1, 2).astype(jnp.float32))
    return out


# speedup vs baseline: 1.1472x; 1.1472x over previous
import functools

import jax
import jax.numpy as jnp
import numpy as np
from jax import lax
from jax.experimental import pallas as pl
from jax.experimental.pallas import tpu as pltpu
from jax.experimental.pallas import tpu_sc as plsc

_V = 1000000
_D = 128
_B = 4096
_L = 50

_NC = 2
_NS = 16
_NW = _NC * _NS
_BPW = _B // _NW
_RPC = 2
_IPC = _RPC * _L
_STRIDE = 104
_CHUNKS = _BPW // _RPC
_LANES = 16
_KV = _D // _LANES


_NBUF = 4


def _pool_body(idx_hbm, rid_hbm, zero_hbm, table_hbm, pooled_hbm,
               idx_v, rid_v, bufs, pooled_v, shared_acc, gsems, ssems):
    cid = lax.axis_index("c")
    sid = lax.axis_index("s")
    wid = sid * _NC + cid
    pltpu.sync_copy(idx_hbm.at[wid], idx_v)
    pltpu.sync_copy(rid_hbm.at[sid], rid_v)
    pltpu.sync_copy(zero_hbm, pooled_v)
    pltpu.sync_copy(pooled_v, shared_acc.at[pl.ds(sid * _BPW, _BPW)])

    def _gather(c, p):
        return pltpu.async_copy(table_hbm.at[idx_v.at[c]], bufs[p], gsems[p])

    _gather(0, 0)
    _gather(1, 1)

    def quad_body(c4, carry):
        for p in range(_NBUF):
            c = c4 * _NBUF + p
            pltpu.make_async_copy(
                table_hbm.at[idx_v.at[c]], bufs[p], gsems[p]
            ).wait()
            pltpu.async_copy(bufs[p], shared_acc.at[rid_v.at[c]], ssems[p],
                             add=True)
            q = (p + 2) % _NBUF
            prev = c - (_NBUF - 2)

            @pl.when(jnp.logical_and(prev >= 0, c + 2 < _CHUNKS))
            def _():
                pltpu.make_async_copy(
                    bufs[q], shared_acc.at[rid_v.at[prev]], ssems[q]
                ).wait()

            @pl.when(c + 2 < _CHUNKS)
            def _():
                _gather(c + 2, q)

        return carry

    lax.fori_loop(0, _CHUNKS // _NBUF, quad_body, 0)
    for p in range(_NBUF):
        c = _CHUNKS - _NBUF + p
        pltpu.make_async_copy(
            bufs[p % _NBUF], shared_acc.at[rid_v.at[c]], ssems[c % _NBUF]
        ).wait()
    pltpu.sync_copy(shared_acc.at[pl.ds(sid * _BPW, _BPW)], pooled_v)
    pltpu.sync_copy(pooled_v, pooled_hbm.at[pl.ds(wid * _BPW, _BPW)])


_pool = functools.partial(
    pl.kernel,
    mesh=plsc.VectorSubcoreMesh(core_axis_name="c", subcore_axis_name="s"),
    out_type=jax.ShapeDtypeStruct((_B, _D), jnp.float32),
    scratch_types=[
        pltpu.VMEM((_CHUNKS, _IPC), jnp.int32),
        pltpu.VMEM((_CHUNKS, _IPC), jnp.int32),
        tuple(pltpu.VMEM((_IPC, _D), jnp.float32) for _ in range(_NBUF)),
        pltpu.VMEM((_BPW, _D), jnp.float32),
        pltpu.VMEM_SHARED((_NS * _BPW, _D), jnp.float32),
        tuple(pltpu.SemaphoreType.DMA for _ in range(_NBUF)),
        tuple(pltpu.SemaphoreType.DMA for _ in range(_NBUF)),
    ],
)(_pool_body)

_RID = (
    np.repeat(np.arange(_BPW, dtype=np.int32).reshape(_CHUNKS, _RPC), _L, axis=1)
    [None, :, :]
    + (np.arange(_NS, dtype=np.int32) * _BPW)[:, None, None]
)


def _linear_body(pooled_ref, wt_ref, bias_ref, out_ref):
    out_ref[...] = (
        jnp.dot(pooled_ref[...], wt_ref[...], preferred_element_type=jnp.float32)
        + bias_ref[...]
    )


def kernel(s, table, W, b):
    s32 = s.astype(jnp.int32)
    flat_idx = s32.reshape(_NW, _CHUNKS, _IPC)
    rid = jnp.asarray(_RID)
    zeros = jnp.zeros((_BPW, _D), jnp.float32)
    pooled = _pool(flat_idx, rid, zeros, table)
    blk = 512
    out = pl.pallas_call(
        _linear_body,
        grid=(_B // blk,),
        in_specs=[
            pl.BlockSpec((blk, _D), lambda i: (i, 0)),
            pl.BlockSpec((_D, 2), lambda i: (0, 0)),
            pl.BlockSpec((1, 2), lambda i: (0, 0)),
        ],
        out_specs=pl.BlockSpec((blk, 2), lambda i: (i, 0)),
        out_shape=jax.ShapeDtypeStruct((_B, 2), jnp.float32),
    )(pooled, W.T.astype(jnp.float32), (_L * b).reshape(1, 2).astype(jnp.float32))
    return out


# speedup vs baseline: 1.1812x; 1.0297x over previous
import functools

import jax
import jax.numpy as jnp
import numpy as np
from jax import lax
from jax.experimental import pallas as pl
from jax.experimental.pallas import tpu as pltpu
from jax.experimental.pallas import tpu_sc as plsc

_V = 1000000
_D = 128
_B = 4096
_L = 50

_NC = 2
_NS = 16
_NW = _NC * _NS
_BPW = _B // _NW
_RPC = 2
_IPC = _RPC * _L
_STRIDE = 104
_CHUNKS = _BPW // _RPC
_LANES = 16
_KV = _D // _LANES


_NBUF = 4


def _pool_body(idx_hbm, rid_hbm, zero_hbm, table_hbm, pooled_hbm,
               idx_v, rid_v, bufs, pooled_v, shared_acc, gsems, ssems):
    cid = lax.axis_index("c")
    sid = lax.axis_index("s")
    wid = sid * _NC + cid
    pltpu.sync_copy(idx_hbm.at[wid], idx_v)
    pltpu.sync_copy(rid_hbm.at[sid], rid_v)
    pltpu.sync_copy(zero_hbm, pooled_v)
    pltpu.sync_copy(pooled_v, shared_acc.at[pl.ds(sid * _BPW, _BPW)])

    def _gather(c, p):
        return pltpu.async_copy(table_hbm.at[idx_v.at[c]], bufs[p], gsems[p])

    _gather(0, 0)
    _gather(1, 1)

    def quad_body(c4, carry):
        for p in range(_NBUF):
            c = c4 * _NBUF + p
            pltpu.make_async_copy(
                table_hbm.at[idx_v.at[c]], bufs[p], gsems[p]
            ).wait()
            pltpu.async_copy(bufs[p], shared_acc.at[rid_v.at[c]], ssems[p],
                             add=True)
            q = (p + 2) % _NBUF
            prev = c - (_NBUF - 2)

            @pl.when(jnp.logical_and(prev >= 0, c + 2 < _CHUNKS))
            def _():
                pltpu.make_async_copy(
                    bufs[q], shared_acc.at[rid_v.at[prev]], ssems[q]
                ).wait()

            @pl.when(c + 2 < _CHUNKS)
            def _():
                _gather(c + 2, q)

        return carry

    lax.fori_loop(0, _CHUNKS // _NBUF, quad_body, 0)
    for p in range(_NBUF):
        c = _CHUNKS - _NBUF + p
        pltpu.make_async_copy(
            bufs[p % _NBUF], shared_acc.at[rid_v.at[c]], ssems[c % _NBUF]
        ).wait()
    pltpu.sync_copy(shared_acc.at[pl.ds(sid * _BPW, _BPW)], pooled_v)
    pltpu.sync_copy(pooled_v, pooled_hbm.at[pl.ds(wid * _BPW, _BPW)])


_pool = functools.partial(
    pl.kernel,
    mesh=plsc.VectorSubcoreMesh(core_axis_name="c", subcore_axis_name="s"),
    out_type=jax.ShapeDtypeStruct((_B, _D), jnp.float32),
    scratch_types=[
        pltpu.VMEM((_CHUNKS, _IPC), jnp.int32),
        pltpu.VMEM((_CHUNKS, _IPC), jnp.int32),
        tuple(pltpu.VMEM((_IPC, _D), jnp.float32) for _ in range(_NBUF)),
        pltpu.VMEM((_BPW, _D), jnp.float32),
        pltpu.VMEM_SHARED((_NS * _BPW, _D), jnp.float32),
        tuple(pltpu.SemaphoreType.DMA for _ in range(_NBUF)),
        tuple(pltpu.SemaphoreType.DMA for _ in range(_NBUF)),
    ],
)(_pool_body)

_RID = (
    np.repeat(np.arange(_BPW, dtype=np.int32).reshape(_CHUNKS, _RPC), _L, axis=1)
    [None, :, :]
    + (np.arange(_NS, dtype=np.int32) * _BPW)[:, None, None]
)


def _linear_body(pooled_ref, wt_ref, bias_ref, out_ref):
    out_ref[...] = (
        jnp.dot(pooled_ref[...], wt_ref[...], preferred_element_type=jnp.float32)
        + bias_ref[...]
    )


def kernel(s, table, W, b):
    s32 = s.astype(jnp.int32)
    flat_idx = s32.reshape(_NW, _CHUNKS, _IPC)
    rid = jnp.asarray(_RID)
    zeros = jnp.zeros((_BPW, _D), jnp.float32)
    pooled = _pool(flat_idx, rid, zeros, table)
    out = pl.pallas_call(
        _linear_body,
        out_shape=jax.ShapeDtypeStruct((_B, 2), jnp.float32),
    )(pooled, W.T.astype(jnp.float32), (_L * b).reshape(1, 2).astype(jnp.float32))
    return out
